# Initial kernel scaffold; baseline (speedup 1.0000x reference)
import jax
import jax.numpy as jnp
from jax.experimental import pallas as pl


def kernel(x, edge_index, W1, b1, W2, b2):
    raise NotImplementedError("write your pallas kernel here")


# speedup vs baseline: 17.7789x; 17.7789x over previous
import functools

import jax
import jax.numpy as jnp
from jax import lax
from jax.experimental import pallas as pl
from jax.experimental.pallas import tpu as pltpu
from jax.experimental.pallas import tpu_sc as plsc

N = 10000
E = 320000
D = 128
K = 10
ALPHA = 0.1

NCORE = 2
NTILE = 16
FH = D // NCORE
NPAD = 10240
RPT = NPAD // NTILE
NPADROWS = NPAD - N
EC = 128
NCHUNK = 160
ETILE = NCHUNK * EC
EPAD = ETILE * NTILE


def _rsqrt16(d):
    t = 0.5 * (d + 1.0)
    for _ in range(20):
        t = 0.5 * (t + d / t)
    return 1.0 / t


def _deg_body(colp, dinv, deg_sh, ic, ones_b, zeros_b, dbuf, ob1):
    c = lax.axis_index("c")
    s = lax.axis_index("s")

    @pl.when(c == 0)
    def _():
        def fill(i, _):
            z16 = jnp.zeros((16,), jnp.float32)
            ones_b[i, :] = z16 + 1.0
            zeros_b[i, :] = z16
            return 0

        lax.fori_loop(0, EC, fill, 0)
        pltpu.sync_copy(colp.at[s], ic)
        for j in range(RPT // EC):
            pltpu.sync_copy(zeros_b, deg_sh.at[pl.ds(s * RPT + j * EC, EC)])
        plsc.subcore_barrier()

        def scat(j, _):
            pltpu.sync_copy(ones_b, deg_sh.at[ic.at[j]], add=True)
            return 0

        lax.fori_loop(0, NCHUNK, scat, 0)
        plsc.subcore_barrier()
        pltpu.sync_copy(deg_sh.at[pl.ds(s * RPT, RPT)], dbuf)

        def conv(r, _):
            d = dbuf[r, :] + 1.0
            ob1[r, :] = _rsqrt16(d)
            return 0

        lax.fori_loop(0, RPT, conv, 0)
        pltpu.sync_copy(ob1, dinv.at[pl.ds(s * RPT, RPT)])


def _make_deg_kernel():
    mesh = plsc.VectorSubcoreMesh(
        core_axis_name="c", subcore_axis_name="s", num_cores=NCORE,
        num_subcores=NTILE,
    )
    return pl.kernel(
        _deg_body,
        out_type=jax.ShapeDtypeStruct((NPAD, 16), jnp.float32),
        mesh=mesh,
        compiler_params=pltpu.CompilerParams(use_tc_tiling_on_sc=False),
        scratch_types=[
            pltpu.VMEM_SHARED((NPAD, 16), jnp.float32),
            pltpu.VMEM((NCHUNK, EC), jnp.int32),
            pltpu.VMEM((EC, 16), jnp.float32),
            pltpu.VMEM((EC, 16), jnp.float32),
            pltpu.VMEM((RPT, 16), jnp.float32),
            pltpu.VMEM((RPT, 16), jnp.float32),
        ],
    )


def _appnp_body(relu, h, dinv, irp, icp, xout, gout,
                agg_sh, ir, ic, m0, m1, ub, dv_s, zb, s0, s1):
    c = lax.axis_index("c")
    s = lax.axis_index("s")
    xo = xout.at[c]
    go = gout.at[c]
    hh = h.at[c]
    r0 = s * RPT

    pltpu.sync_copy(irp.at[s], ir)
    pltpu.sync_copy(icp.at[s], ic)
    pltpu.sync_copy(dinv.at[pl.ds(r0, RPT)], dv_s)

    def zfill(i, _):
        for f in range(4):
            zb[i, pl.ds(f * 16, 16)] = jnp.zeros((16,), jnp.float32)
        return 0

    lax.fori_loop(0, EC, zfill, 0)
    for j in range(RPT // EC):
        pltpu.sync_copy(zb, agg_sh.at[pl.ds(r0 + j * EC, EC)])

    for j in range(RPT // EC):
        pltpu.sync_copy(hh.at[pl.ds(r0 + j * EC, EC)], m0)

        def ibody(r, _, j=j):
            dv = dv_s[j * EC + r, :]
            for f in range(4):
                hv = m0[r, pl.ds(f * 16, 16)]
                z0 = dv * hv
                m1[r, pl.ds(f * 16, 16)] = z0
                ub[r, pl.ds(f * 16, 16)] = ALPHA * z0
            return 0

        lax.fori_loop(0, EC, ibody, 0)
        pltpu.sync_copy(m1, xo.at[pl.ds(r0 + j * EC, EC)])
        pltpu.sync_copy(ub, go.at[pl.ds(r0 + j * EC, EC)])
    plsc.subcore_barrier()

    def scatter_phase():
        pltpu.async_copy(xo.at[ir.at[0]], m0, s0)

        def jbody(j, _):
            c0 = 2 * j
            c1 = 2 * j + 1
            pltpu.async_copy(xo.at[ir.at[c1]], m1, s1)
            pltpu.make_async_copy(xo.at[ir.at[c0]], m0, s0).wait()
            pltpu.sync_copy(m0, agg_sh.at[ic.at[c0]], add=True)

            @pl.when(c1 + 1 < NCHUNK)
            def _():
                pltpu.async_copy(xo.at[ir.at[c1 + 1]], m0, s0)

            pltpu.make_async_copy(xo.at[ir.at[c1]], m1, s1).wait()
            pltpu.sync_copy(m1, agg_sh.at[ic.at[c1]], add=True)
            return 0

        lax.fori_loop(0, NCHUNK // 2, jbody, 0)

    def update_phase(last):
        for j in range(RPT // EC):
            rb = r0 + j * EC
            pltpu.sync_copy(agg_sh.at[pl.ds(rb, EC)], m0)
            pltpu.sync_copy(xo.at[pl.ds(rb, EC)], m1)
            pltpu.sync_copy(go.at[pl.ds(rb, EC)], ub)

            def ubody(r, _, j=j):
                li = j * EC + r
                dv = dv_s[li, :]
                qc = (1.0 - ALPHA) * dv * dv
                if last:
                    sd = 1.0 / dv
                for f in range(4):
                    a = m0[r, pl.ds(f * 16, 16)]
                    z = m1[r, pl.ds(f * 16, 16)]
                    g = ub[r, pl.ds(f * 16, 16)]
                    zn = qc * (a + z) + g
                    if last:
                        zn = zn * sd
                        if relu:
                            zn = jnp.maximum(zn, 0.0)
                    ub[r, pl.ds(f * 16, 16)] = zn
                return 0

            lax.fori_loop(0, EC, ubody, 0)
            pltpu.sync_copy(ub, xo.at[pl.ds(rb, EC)])
            pltpu.sync_copy(zb, agg_sh.at[pl.ds(rb, EC)])

    for t in range(K):
        scatter_phase()
        plsc.subcore_barrier()
        update_phase(last=(t == K - 1))
        plsc.subcore_barrier()


def _make_appnp_kernel(relu):
    mesh = plsc.VectorSubcoreMesh(
        core_axis_name="c", subcore_axis_name="s", num_cores=NCORE,
        num_subcores=NTILE,
    )
    return pl.kernel(
        functools.partial(_appnp_body, relu),
        out_type=(
            jax.ShapeDtypeStruct((NCORE, NPAD, FH), jnp.float32),
            jax.ShapeDtypeStruct((NCORE, NPAD, FH), jnp.float32),
        ),
        mesh=mesh,
        compiler_params=pltpu.CompilerParams(use_tc_tiling_on_sc=False),
        scratch_types=[
            pltpu.VMEM_SHARED((NPAD, FH), jnp.float32),
            pltpu.VMEM((NCHUNK, EC), jnp.int32),
            pltpu.VMEM((NCHUNK, EC), jnp.int32),
            pltpu.VMEM((EC, FH), jnp.float32),
            pltpu.VMEM((EC, FH), jnp.float32),
            pltpu.VMEM((EC, FH), jnp.float32),
            pltpu.VMEM((RPT, 16), jnp.float32),
            pltpu.VMEM((EC, FH), jnp.float32),
            pltpu.SemaphoreType.DMA,
            pltpu.SemaphoreType.DMA,
        ],
    )


def _mm_body(x_ref, wt_ref, b_ref, o_ref):
    o_ref[...] = (
        jnp.dot(x_ref[...], wt_ref[...], preferred_element_type=jnp.float32)
        + b_ref[0][None, :]
    )


def _linear(x, w, b):
    bm = 1000
    wt = w.T
    b8 = jnp.broadcast_to(b[None, :], (8, w.shape[0]))
    return pl.pallas_call(
        _mm_body,
        grid=(N // bm,),
        in_specs=[
            pl.BlockSpec((bm, x.shape[1]), lambda i: (i, 0)),
            pl.BlockSpec(wt.shape, lambda i: (0, 0)),
            pl.BlockSpec(b8.shape, lambda i: (0, 0)),
        ],
        out_specs=pl.BlockSpec((bm, w.shape[0]), lambda i: (i, 0)),
        out_shape=jax.ShapeDtypeStruct((N, w.shape[0]), jnp.float32),
    )(x, wt, b8)


def _split(h):
    hp = jnp.pad(h, ((0, NPAD - N), (0, 0)))
    return hp.reshape(NPAD, NCORE, FH).transpose(1, 0, 2)


def _unsplit(o):
    return o.transpose(1, 0, 2).reshape(NPAD, D)[:N]


def kernel(x, edge_index, W1, b1, W2, b2):
    ei = edge_index.astype(jnp.int32)
    row, col = ei[0], ei[1]
    npad_e = EPAD - E
    pad_idx = (jnp.arange(npad_e, dtype=jnp.int32) % NPADROWS) + N
    irp = jnp.concatenate([row, pad_idx]).reshape(NTILE, NCHUNK, EC)
    icp = jnp.concatenate([col, pad_idx]).reshape(NTILE, NCHUNK, EC)

    dinv = _make_deg_kernel()(icp)

    appnp_r = _make_appnp_kernel(True)
    appnp_n = _make_appnp_kernel(False)

    h1 = _linear(x, W1, b1)
    o1, _g1 = appnp_r(_split(h1), dinv, irp, icp)
    o2, _g2 = appnp_r(o1, dinv, irp, icp)
    h2 = _linear(_unsplit(o2), W2, b2)
    o3, _g3 = appnp_n(_split(h2), dinv, irp, icp)
    return _unsplit(o3)


# speedup vs baseline: 19.1020x; 1.0744x over previous
import functools

import jax
import jax.numpy as jnp
from jax import lax
from jax.experimental import pallas as pl
from jax.experimental.pallas import tpu as pltpu
from jax.experimental.pallas import tpu_sc as plsc

N = 10000
E = 320000
D = 128
K = 10
ALPHA = 0.1

NCORE = 2
NTILE = 16
FH = D // NCORE
NPAD = 10240
RPT = NPAD // NTILE
NPADROWS = NPAD - N
EC = 128
NCHUNK = 160
ETILE = NCHUNK * EC
EPAD = ETILE * NTILE


def _rsqrt16(d):
    t = 0.5 * (d + 1.0)
    for _ in range(20):
        t = 0.5 * (t + d / t)
    return 1.0 / t


def _deg_body(colp, dinv, deg_sh, ic, ones_b, zeros_b, dbuf, ob1):
    c = lax.axis_index("c")
    s = lax.axis_index("s")

    @pl.when(c == 0)
    def _():
        def fill(i, _):
            z16 = jnp.zeros((16,), jnp.float32)
            ones_b[i, :] = z16 + 1.0
            zeros_b[i, :] = z16
            return 0

        lax.fori_loop(0, EC, fill, 0)
        pltpu.sync_copy(colp.at[s], ic)
        for j in range(RPT // EC):
            pltpu.sync_copy(zeros_b, deg_sh.at[pl.ds(s * RPT + j * EC, EC)])
        plsc.subcore_barrier()

        def scat(j, _):
            pltpu.sync_copy(ones_b, deg_sh.at[ic.at[j]], add=True)
            return 0

        lax.fori_loop(0, NCHUNK, scat, 0)
        plsc.subcore_barrier()
        pltpu.sync_copy(deg_sh.at[pl.ds(s * RPT, RPT)], dbuf)

        def conv(r, _):
            d = dbuf[r, :] + 1.0
            ob1[r, :] = _rsqrt16(d)
            return 0

        lax.fori_loop(0, RPT, conv, 0)
        pltpu.sync_copy(ob1, dinv.at[pl.ds(s * RPT, RPT)])


def _make_deg_kernel():
    mesh = plsc.VectorSubcoreMesh(
        core_axis_name="c", subcore_axis_name="s", num_cores=NCORE,
        num_subcores=NTILE,
    )
    return pl.kernel(
        _deg_body,
        out_type=jax.ShapeDtypeStruct((NPAD, 16), jnp.float32),
        mesh=mesh,
        compiler_params=pltpu.CompilerParams(use_tc_tiling_on_sc=False),
        scratch_types=[
            pltpu.VMEM_SHARED((NPAD, 16), jnp.float32),
            pltpu.VMEM((NCHUNK, EC), jnp.int32),
            pltpu.VMEM((EC, 16), jnp.float32),
            pltpu.VMEM((EC, 16), jnp.float32),
            pltpu.VMEM((RPT, 16), jnp.float32),
            pltpu.VMEM((RPT, 16), jnp.float32),
        ],
    )


NBUF = 4


def _appnp_body(relu, h, dinv, irp, icp, xout, gout,
                agg_sh, ir, ic, m0, m1, m2, m3, dvc, zb, *sems):
    c = lax.axis_index("c")
    s = lax.axis_index("s")
    xo = xout.at[c]
    go = gout.at[c]
    hh = h.at[c]
    r0 = s * RPT
    m = (m0, m1, m2, m3)
    gsem = sems[:NBUF]
    ssem = sems[NBUF:]

    pltpu.sync_copy(irp.at[s], ir)
    pltpu.sync_copy(icp.at[s], ic)

    def zfill(i, _):
        for f in range(4):
            zb[i, pl.ds(f * 16, 16)] = jnp.zeros((16,), jnp.float32)
        return 0

    lax.fori_loop(0, EC, zfill, 0)
    for j in range(RPT // EC):
        pltpu.sync_copy(zb, agg_sh.at[pl.ds(r0 + j * EC, EC)])

    for j in range(RPT // EC):
        rb = r0 + j * EC
        pltpu.sync_copy(hh.at[pl.ds(rb, EC)], m0)
        pltpu.sync_copy(dinv.at[pl.ds(rb, EC)], dvc)

        def ibody(r, _):
            dv = dvc[r, :]
            for f in range(4):
                hv = m0[r, pl.ds(f * 16, 16)]
                z0 = dv * hv
                m1[r, pl.ds(f * 16, 16)] = z0
                m2[r, pl.ds(f * 16, 16)] = ALPHA * z0
            return 0

        lax.fori_loop(0, EC, ibody, 0)
        pltpu.sync_copy(m1, xo.at[pl.ds(rb, EC)])
        pltpu.sync_copy(m2, go.at[pl.ds(rb, EC)])
    plsc.subcore_barrier()

    def scatter_phase():
        for b in range(NBUF):
            pltpu.async_copy(xo.at[ir.at[b]], m[b], gsem[b])

        def jbody(j, _):
            for b in range(NBUF):
                cc = NBUF * j + b
                pltpu.make_async_copy(xo.at[ir.at[cc]], m[b], gsem[b]).wait()
                pltpu.async_copy(m[b], agg_sh.at[ic.at[cc]], ssem[b],
                                 add=True)
            for b in range(NBUF):
                cc = NBUF * j + b

                @pl.when(cc + NBUF < NCHUNK)
                def _(b=b, cc=cc):
                    pltpu.make_async_copy(
                        m[b], agg_sh.at[ic.at[cc]], ssem[b]).wait()
                    pltpu.async_copy(xo.at[ir.at[cc + NBUF]], m[b], gsem[b])
            return 0

        lax.fori_loop(0, NCHUNK // NBUF, jbody, 0)
        for b in range(NBUF):
            cc = NCHUNK - NBUF + b
            pltpu.make_async_copy(m[b], agg_sh.at[ic.at[cc]], ssem[b]).wait()

    def update_phase(last):
        for j in range(RPT // EC):
            rb = r0 + j * EC
            pltpu.sync_copy(agg_sh.at[pl.ds(rb, EC)], m0)
            pltpu.sync_copy(xo.at[pl.ds(rb, EC)], m1)
            pltpu.sync_copy(go.at[pl.ds(rb, EC)], m2)
            pltpu.sync_copy(dinv.at[pl.ds(rb, EC)], dvc)

            def ubody(r, _):
                dv = dvc[r, :]
                qc = (1.0 - ALPHA) * dv * dv
                if last:
                    sd = 1.0 / dv
                for f in range(4):
                    a = m0[r, pl.ds(f * 16, 16)]
                    z = m1[r, pl.ds(f * 16, 16)]
                    g = m2[r, pl.ds(f * 16, 16)]
                    zn = qc * (a + z) + g
                    if last:
                        zn = zn * sd
                        if relu:
                            zn = jnp.maximum(zn, 0.0)
                    m3[r, pl.ds(f * 16, 16)] = zn
                return 0

            lax.fori_loop(0, EC, ubody, 0)
            pltpu.sync_copy(m3, xo.at[pl.ds(rb, EC)])
            pltpu.sync_copy(zb, agg_sh.at[pl.ds(rb, EC)])

    for t in range(K):
        scatter_phase()
        plsc.subcore_barrier()
        update_phase(last=(t == K - 1))
        plsc.subcore_barrier()


def _make_appnp_kernel(relu):
    mesh = plsc.VectorSubcoreMesh(
        core_axis_name="c", subcore_axis_name="s", num_cores=NCORE,
        num_subcores=NTILE,
    )
    return pl.kernel(
        functools.partial(_appnp_body, relu),
        out_type=(
            jax.ShapeDtypeStruct((NCORE, NPAD, FH), jnp.float32),
            jax.ShapeDtypeStruct((NCORE, NPAD, FH), jnp.float32),
        ),
        mesh=mesh,
        compiler_params=pltpu.CompilerParams(use_tc_tiling_on_sc=False),
        scratch_types=(
            [
                pltpu.VMEM_SHARED((NPAD, FH), jnp.float32),
                pltpu.VMEM((NCHUNK, EC), jnp.int32),
                pltpu.VMEM((NCHUNK, EC), jnp.int32),
            ]
            + [pltpu.VMEM((EC, FH), jnp.float32) for _ in range(NBUF)]
            + [
                pltpu.VMEM((EC, 16), jnp.float32),
                pltpu.VMEM((EC, FH), jnp.float32),
            ]
            + [pltpu.SemaphoreType.DMA for _ in range(2 * NBUF)]
        ),
    )


def _mm_body(x_ref, wt_ref, b_ref, o_ref):
    o_ref[...] = (
        jnp.dot(x_ref[...], wt_ref[...], preferred_element_type=jnp.float32)
        + b_ref[0][None, :]
    )


def _linear(x, w, b):
    bm = 1000
    wt = w.T
    b8 = jnp.broadcast_to(b[None, :], (8, w.shape[0]))
    return pl.pallas_call(
        _mm_body,
        grid=(N // bm,),
        in_specs=[
            pl.BlockSpec((bm, x.shape[1]), lambda i: (i, 0)),
            pl.BlockSpec(wt.shape, lambda i: (0, 0)),
            pl.BlockSpec(b8.shape, lambda i: (0, 0)),
        ],
        out_specs=pl.BlockSpec((bm, w.shape[0]), lambda i: (i, 0)),
        out_shape=jax.ShapeDtypeStruct((N, w.shape[0]), jnp.float32),
    )(x, wt, b8)


def _split(h):
    hp = jnp.pad(h, ((0, NPAD - N), (0, 0)))
    return hp.reshape(NPAD, NCORE, FH).transpose(1, 0, 2)


def _unsplit(o):
    return o.transpose(1, 0, 2).reshape(NPAD, D)[:N]


def kernel(x, edge_index, W1, b1, W2, b2):
    ei = edge_index.astype(jnp.int32)
    row, col = ei[0], ei[1]
    npad_e = EPAD - E
    pad_idx = (jnp.arange(npad_e, dtype=jnp.int32) % NPADROWS) + N
    irp = jnp.concatenate([row, pad_idx]).reshape(NTILE, NCHUNK, EC)
    icp = jnp.concatenate([col, pad_idx]).reshape(NTILE, NCHUNK, EC)

    dinv = _make_deg_kernel()(icp)

    appnp_r = _make_appnp_kernel(True)
    appnp_n = _make_appnp_kernel(False)

    h1 = _linear(x, W1, b1)
    o1, _g1 = appnp_r(_split(h1), dinv, irp, icp)
    o2, _g2 = appnp_r(o1, dinv, irp, icp)
    h2 = _linear(_unsplit(o2), W2, b2)
    o3, _g3 = appnp_n(_split(h2), dinv, irp, icp)
    return _unsplit(o3)


# speedup vs baseline: 21.0024x; 1.0995x over previous
import functools

import jax
import jax.numpy as jnp
from jax import lax
from jax.experimental import pallas as pl
from jax.experimental.pallas import tpu as pltpu
from jax.experimental.pallas import tpu_sc as plsc

N = 10000
E = 320000
D = 128
K = 10
ALPHA = 0.1

NCORE = 2
NTILE = 16
FH = D // NCORE
NPAD = 10240
RPT = NPAD // NTILE
NPADROWS = NPAD - N
EC = 128
NCHUNK = 160
ETILE = NCHUNK * EC
EPAD = ETILE * NTILE


def _rsqrt16(d):
    t = 0.5 * (d + 1.0)
    for _ in range(20):
        t = 0.5 * (t + d / t)
    return 1.0 / t


def _deg_body(colp, dinv, deg_sh, ic, ones_b, zeros_b, dbuf, ob1):
    c = lax.axis_index("c")
    s = lax.axis_index("s")

    @pl.when(c == 0)
    def _():
        def fill(i, _):
            z16 = jnp.zeros((16,), jnp.float32)
            ones_b[i, :] = z16 + 1.0
            zeros_b[i, :] = z16
            return 0

        lax.fori_loop(0, EC, fill, 0)
        pltpu.sync_copy(colp.at[s], ic)
        for j in range(RPT // EC):
            pltpu.sync_copy(zeros_b, deg_sh.at[pl.ds(s * RPT + j * EC, EC)])
        plsc.subcore_barrier()

        def scat(j, _):
            pltpu.sync_copy(ones_b, deg_sh.at[ic.at[j]], add=True)
            return 0

        lax.fori_loop(0, NCHUNK, scat, 0)
        plsc.subcore_barrier()
        pltpu.sync_copy(deg_sh.at[pl.ds(s * RPT, RPT)], dbuf)

        def conv(r, _):
            d = dbuf[r, :] + 1.0
            ob1[r, :] = _rsqrt16(d)
            return 0

        lax.fori_loop(0, RPT, conv, 0)
        pltpu.sync_copy(ob1, dinv.at[pl.ds(s * RPT, RPT)])


def _make_deg_kernel():
    mesh = plsc.VectorSubcoreMesh(
        core_axis_name="c", subcore_axis_name="s", num_cores=NCORE,
        num_subcores=NTILE,
    )
    return pl.kernel(
        _deg_body,
        out_type=jax.ShapeDtypeStruct((NPAD, 16), jnp.float32),
        mesh=mesh,
        compiler_params=pltpu.CompilerParams(use_tc_tiling_on_sc=False),
        scratch_types=[
            pltpu.VMEM_SHARED((NPAD, 16), jnp.float32),
            pltpu.VMEM((NCHUNK, EC), jnp.int32),
            pltpu.VMEM((EC, 16), jnp.float32),
            pltpu.VMEM((EC, 16), jnp.float32),
            pltpu.VMEM((RPT, 16), jnp.float32),
            pltpu.VMEM((RPT, 16), jnp.float32),
        ],
    )


NBUF = 4


def _appnp_body(relu, h, dinv, irp, icp, xout, gout,
                agg_sh, ir, ic, m0, m1, m2, m3, dvc, zb, *sems):
    c = lax.axis_index("c")
    s = lax.axis_index("s")
    xo = xout.at[c]
    go = gout.at[c]
    hh = h.at[c]
    r0 = s * RPT
    m = (m0, m1, m2, m3)
    gsem = sems[:NBUF]
    ssem = sems[NBUF:]

    pltpu.sync_copy(irp.at[s], ir)
    pltpu.sync_copy(icp.at[s], ic)

    def zfill(i, _):
        for f in range(4):
            zb[i, pl.ds(f * 16, 16)] = jnp.zeros((16,), jnp.float32)
        return 0

    lax.fori_loop(0, EC, zfill, 0)
    for j in range(RPT // EC):
        pltpu.sync_copy(zb, agg_sh.at[pl.ds(r0 + j * EC, EC)])

    for j in range(RPT // EC):
        rb = r0 + j * EC
        pltpu.sync_copy(hh.at[pl.ds(rb, EC)], m0)
        pltpu.sync_copy(dinv.at[pl.ds(rb, EC)], dvc)

        def ibody(r, _):
            dv = dvc[r, :]
            for f in range(4):
                hv = m0[r, pl.ds(f * 16, 16)]
                z0 = dv * hv
                m1[r, pl.ds(f * 16, 16)] = z0
                m2[r, pl.ds(f * 16, 16)] = ALPHA * z0
            return 0

        lax.fori_loop(0, EC, ibody, 0)
        pltpu.sync_copy(m1, xo.at[pl.ds(rb, EC)])
        pltpu.sync_copy(m2, go.at[pl.ds(rb, EC)])
    plsc.subcore_barrier()

    def scatter_phase():
        for b in range(NBUF):
            pltpu.async_copy(xo.at[ir.at[b]], m[b], gsem[b])

        def jbody(j, _):
            for b in range(NBUF):
                cc = NBUF * j + b
                pltpu.make_async_copy(xo.at[ir.at[cc]], m[b], gsem[b]).wait()
                pltpu.async_copy(m[b], agg_sh.at[ic.at[cc]], ssem[b],
                                 add=True)
            for b in range(NBUF):
                cc = NBUF * j + b

                @pl.when(cc + NBUF < NCHUNK)
                def _(b=b, cc=cc):
                    pltpu.make_async_copy(
                        m[b], agg_sh.at[ic.at[cc]], ssem[b]).wait()
                    pltpu.async_copy(xo.at[ir.at[cc + NBUF]], m[b], gsem[b])
            return 0

        lax.fori_loop(0, NCHUNK // NBUF, jbody, 0)
        for b in range(NBUF):
            cc = NCHUNK - NBUF + b
            pltpu.make_async_copy(m[b], agg_sh.at[ic.at[cc]], ssem[b]).wait()

    def update_phase(last):
        for j in range(RPT // EC):
            rb = r0 + j * EC
            pltpu.async_copy(agg_sh.at[pl.ds(rb, EC)], m0, gsem[0])
            pltpu.async_copy(xo.at[pl.ds(rb, EC)], m1, gsem[1])
            pltpu.async_copy(go.at[pl.ds(rb, EC)], m2, gsem[2])
            pltpu.async_copy(dinv.at[pl.ds(rb, EC)], dvc, gsem[3])
            if j > 0:
                rp = r0 + (j - 1) * EC
                pltpu.make_async_copy(m3, xo.at[pl.ds(rp, EC)], ssem[0]).wait()
                pltpu.make_async_copy(zb, agg_sh.at[pl.ds(rp, EC)], ssem[1]).wait()
            pltpu.make_async_copy(agg_sh.at[pl.ds(rb, EC)], m0, gsem[0]).wait()
            pltpu.make_async_copy(xo.at[pl.ds(rb, EC)], m1, gsem[1]).wait()
            pltpu.make_async_copy(go.at[pl.ds(rb, EC)], m2, gsem[2]).wait()
            pltpu.make_async_copy(dinv.at[pl.ds(rb, EC)], dvc, gsem[3]).wait()

            def ubody(r, _):
                dv = dvc[r, :]
                qc = (1.0 - ALPHA) * dv * dv
                if last:
                    sd = 1.0 / dv
                for f in range(4):
                    a = m0[r, pl.ds(f * 16, 16)]
                    z = m1[r, pl.ds(f * 16, 16)]
                    g = m2[r, pl.ds(f * 16, 16)]
                    zn = qc * (a + z) + g
                    if last:
                        zn = zn * sd
                        if relu:
                            zn = jnp.maximum(zn, 0.0)
                    m3[r, pl.ds(f * 16, 16)] = zn
                return 0

            lax.fori_loop(0, EC, ubody, 0)
            pltpu.async_copy(m3, xo.at[pl.ds(rb, EC)], ssem[0])
            pltpu.async_copy(zb, agg_sh.at[pl.ds(rb, EC)], ssem[1])
        rl = r0 + (RPT // EC - 1) * EC
        pltpu.make_async_copy(m3, xo.at[pl.ds(rl, EC)], ssem[0]).wait()
        pltpu.make_async_copy(zb, agg_sh.at[pl.ds(rl, EC)], ssem[1]).wait()

    for t in range(K):
        scatter_phase()
        plsc.subcore_barrier()
        update_phase(last=(t == K - 1))
        plsc.subcore_barrier()


def _make_appnp_kernel(relu):
    mesh = plsc.VectorSubcoreMesh(
        core_axis_name="c", subcore_axis_name="s", num_cores=NCORE,
        num_subcores=NTILE,
    )
    return pl.kernel(
        functools.partial(_appnp_body, relu),
        out_type=(
            jax.ShapeDtypeStruct((NCORE, NPAD, FH), jnp.float32),
            jax.ShapeDtypeStruct((NCORE, NPAD, FH), jnp.float32),
        ),
        mesh=mesh,
        compiler_params=pltpu.CompilerParams(use_tc_tiling_on_sc=False),
        scratch_types=(
            [
                pltpu.VMEM_SHARED((NPAD, FH), jnp.float32),
                pltpu.VMEM((NCHUNK, EC), jnp.int32),
                pltpu.VMEM((NCHUNK, EC), jnp.int32),
            ]
            + [pltpu.VMEM((EC, FH), jnp.float32) for _ in range(NBUF)]
            + [
                pltpu.VMEM((EC, 16), jnp.float32),
                pltpu.VMEM((EC, FH), jnp.float32),
            ]
            + [pltpu.SemaphoreType.DMA for _ in range(2 * NBUF)]
        ),
    )


def _mm_body(x_ref, wt_ref, b_ref, o_ref):
    o_ref[...] = (
        jnp.dot(x_ref[...], wt_ref[...], preferred_element_type=jnp.float32)
        + b_ref[0][None, :]
    )


def _linear(x, w, b):
    bm = 1000
    wt = w.T
    b8 = jnp.broadcast_to(b[None, :], (8, w.shape[0]))
    return pl.pallas_call(
        _mm_body,
        grid=(N // bm,),
        in_specs=[
            pl.BlockSpec((bm, x.shape[1]), lambda i: (i, 0)),
            pl.BlockSpec(wt.shape, lambda i: (0, 0)),
            pl.BlockSpec(b8.shape, lambda i: (0, 0)),
        ],
        out_specs=pl.BlockSpec((bm, w.shape[0]), lambda i: (i, 0)),
        out_shape=jax.ShapeDtypeStruct((N, w.shape[0]), jnp.float32),
    )(x, wt, b8)


def _split(h):
    hp = jnp.pad(h, ((0, NPAD - N), (0, 0)))
    return hp.reshape(NPAD, NCORE, FH).transpose(1, 0, 2)


def _unsplit(o):
    return o.transpose(1, 0, 2).reshape(NPAD, D)[:N]


def kernel(x, edge_index, W1, b1, W2, b2):
    ei = edge_index.astype(jnp.int32)
    row, col = ei[0], ei[1]
    npad_e = EPAD - E
    pad_idx = (jnp.arange(npad_e, dtype=jnp.int32) % NPADROWS) + N
    irp = jnp.concatenate([row, pad_idx]).reshape(NTILE, NCHUNK, EC)
    icp = jnp.concatenate([col, pad_idx]).reshape(NTILE, NCHUNK, EC)

    dinv = _make_deg_kernel()(icp)

    appnp_r = _make_appnp_kernel(True)
    appnp_n = _make_appnp_kernel(False)

    h1 = _linear(x, W1, b1)
    o1, _g1 = appnp_r(_split(h1), dinv, irp, icp)
    o2, _g2 = appnp_r(o1, dinv, irp, icp)
    h2 = _linear(_unsplit(o2), W2, b2)
    o3, _g3 = appnp_n(_split(h2), dinv, irp, icp)
    return _unsplit(o3)
